# baseline (speedup 1.0000x reference)
import jax
import jax.numpy as jnp
from jax.experimental import pallas as pl
from jax.experimental.pallas import tpu as pltpu

_HIDDEN = 64
_N = 128
_ITERS = 2
_BB = 64
_G = 8
_LOG2E = 1.4426950408889634
_NFEAT = 7


def _dot(a, b):
    return jax.lax.dot_general(a, b, (((1,), (0,)), ((), ())),
                               preferred_element_type=jnp.float32)


def _bf(v):
    return v.astype(jnp.bfloat16)


def _mpnn_kernel(x_ref, Wemb_ref, bemb_ref, Wadj_ref, Wmsg_ref, bmsg_ref,
                 Wz_ref, Uz_ref, bz_ref, Wr_ref, Ur_ref, br_ref,
                 Wh_ref, Uh_ref, bh_ref, Wro_ref, bro_ref,
                 out_ref, A_ref):
    H = _HIDDEN
    beff = bemb_ref[...] + _bf(Wemb_ref[_NFEAT:, :]).astype(jnp.float32)
    h = jnp.tanh(_dot(_bf(x_ref[...]), _bf(Wemb_ref[: _NFEAT, :])) + beff)
    ones_blk = jnp.ones((_N, H), jnp.bfloat16)
    for t in range(_ITERS):
        Wap16 = _bf(jnp.concatenate(
            [Wadj_ref[t] * jnp.float32(_LOG2E / (float(H) ** 0.5)),
             Wmsg_ref[t]], axis=1))
        Wzr16 = _bf(0.5 * jnp.concatenate(
            [jnp.concatenate([Wz_ref[t], Wr_ref[t]], axis=1),
             jnp.concatenate([Uz_ref[t], Ur_ref[t]], axis=1)], axis=0))
        bzr = 0.5 * jnp.concatenate([bz_ref[t], br_ref[t]], axis=1)
        Wcand16 = _bf(jnp.concatenate([Wh_ref[t], Uh_ref[t]], axis=0))

        h16 = _bf(h)
        hp = _dot(h16, Wap16)
        hw16 = _bf(hp[:, :H])
        pre16 = _bf(hp[:, H:] + bmsg_ref[t])
        msgs = []
        for g in range(0, _BB, _G):
            es = []
            for b in range(g, g + _G):
                sl = slice(b * _N, (b + 1) * _N)
                logits = jax.lax.dot_general(
                    hw16[sl, :], h16[sl, :], (((1,), (1,)), ((), ())),
                    preferred_element_type=jnp.float32)
                es.append(jnp.exp2(logits))
            ss = []
            for i, b in enumerate(range(g, g + _G)):
                sl = slice(b * _N, (b + 1) * _N)
                pre_aug = jnp.concatenate([pre16[sl, :], ones_blk], axis=1)
                ss.append(_dot(_bf(es[i]), pre_aug))
            for i, b in enumerate(range(g, g + _G)):
                inv = 1.0 / ss[i][:, H:]
                msgs.append(ss[i][:, :H] * inv)
                if t == _ITERS - 1:
                    A_ref[b] = es[i] * jnp.concatenate([inv, inv], axis=1)
        msg = jnp.tanh(jnp.concatenate(msgs, axis=0))
        msg16 = _bf(msg)
        mh16 = jnp.concatenate([msg16, h16], axis=1)
        zr = jnp.tanh(_dot(mh16, Wzr16) + bzr)
        z = 0.5 + 0.5 * zr[:, :H]
        r = 0.5 + 0.5 * zr[:, H:]
        mrh16 = jnp.concatenate([msg16, _bf(r * h)], axis=1)
        htil = jnp.tanh(_dot(mrh16, Wcand16) + bh_ref[t])
        h = h + z * (htil - h)
    pooled = jnp.concatenate(
        [jnp.sum(h[b * _N:(b + 1) * _N, :], axis=0, keepdims=True)
         for b in range(_BB)], axis=0)
    out_ref[...] = jnp.tanh(_dot(_bf(pooled), _bf(Wro_ref[...])) + bro_ref[...])


def kernel(jets, W_emb, b_emb, W_adj, W_msg, b_msg,
           Wz, Uz, bz, Wr, Ur, br, Wh, Uh, bh, W_ro, b_ro):
    B, N, F = jets.shape
    H = _HIDDEN
    x = jets.reshape(B * N, F)

    def rep(ix):
        return pl.BlockSpec(ix, lambda i: (0,) * len(ix))

    out, A = pl.pallas_call(
        _mpnn_kernel,
        grid=(B // _BB,),
        in_specs=[
            pl.BlockSpec((_BB * N, F), lambda i: (i, 0)),
            rep((F + 1, H)),
            rep((1, H)),
            rep((_ITERS, H, H)),
            rep((_ITERS, H, H)),
            rep((_ITERS, 1, H)),
            rep((_ITERS, H, H)), rep((_ITERS, H, H)), rep((_ITERS, 1, H)),
            rep((_ITERS, H, H)), rep((_ITERS, H, H)), rep((_ITERS, 1, H)),
            rep((_ITERS, H, H)), rep((_ITERS, H, H)), rep((_ITERS, 1, H)),
            rep((H, H)),
            rep((1, H)),
        ],
        out_specs=[
            pl.BlockSpec((_BB, H), lambda i: (i, 0)),
            pl.BlockSpec((_BB, N, N), lambda i: (i, 0, 0)),
        ],
        out_shape=[
            jax.ShapeDtypeStruct((B, H), jnp.float32),
            jax.ShapeDtypeStruct((B, N, N), jnp.float32),
        ],
        compiler_params=pltpu.CompilerParams(
            dimension_semantics=("parallel",)),
    )(x, W_emb, b_emb.reshape(1, H),
      W_adj, W_msg, b_msg.reshape(_ITERS, 1, H),
      Wz, Uz, bz.reshape(_ITERS, 1, H),
      Wr, Ur, br.reshape(_ITERS, 1, H),
      Wh, Uh, bh.reshape(_ITERS, 1, H),
      W_ro, b_ro.reshape(1, H))
    return (out, A)


# speedup vs baseline: 1.0392x; 1.0392x over previous
import jax
import jax.numpy as jnp
from jax.experimental import pallas as pl
from jax.experimental.pallas import tpu as pltpu

_HIDDEN = 64
_N = 128
_ITERS = 2
_BB = 64
_G = _BB
_LOG2E = 1.4426950408889634
_NFEAT = 7


def _dot(a, b):
    return jax.lax.dot_general(a, b, (((1,), (0,)), ((), ())),
                               preferred_element_type=jnp.float32)


def _bf(v):
    return v.astype(jnp.bfloat16)


def _mpnn_kernel(x_ref, Wemb_ref, bemb_ref, Wadj_ref, Wmsg_ref, bmsg_ref,
                 Wz_ref, Uz_ref, bz_ref, Wr_ref, Ur_ref, br_ref,
                 Wh_ref, Uh_ref, bh_ref, Wro_ref, bro_ref,
                 out_ref, A_ref):
    H = _HIDDEN
    beff = bemb_ref[...] + _bf(Wemb_ref[_NFEAT:, :]).astype(jnp.float32)
    h = jnp.tanh(_dot(_bf(x_ref[...]), _bf(Wemb_ref[: _NFEAT, :])) + beff)
    ones_blk = jnp.ones((_N, H), jnp.bfloat16)
    for t in range(_ITERS):
        Wap16 = _bf(jnp.concatenate(
            [Wadj_ref[t] * jnp.float32(_LOG2E / (float(H) ** 0.5)),
             Wmsg_ref[t]], axis=1))
        Wzr16 = _bf(0.5 * jnp.concatenate(
            [jnp.concatenate([Wz_ref[t], Wr_ref[t]], axis=1),
             jnp.concatenate([Uz_ref[t], Ur_ref[t]], axis=1)], axis=0))
        bzr = 0.5 * jnp.concatenate([bz_ref[t], br_ref[t]], axis=1)
        Wcand16 = _bf(jnp.concatenate([Wh_ref[t], Uh_ref[t]], axis=0))

        h16 = _bf(h)
        hp = _dot(h16, Wap16)
        hw16 = _bf(hp[:, :H])
        pre16 = _bf(hp[:, H:] + bmsg_ref[t])
        msgs = []
        for g in range(0, _BB, _G):
            es = []
            for b in range(g, g + _G):
                sl = slice(b * _N, (b + 1) * _N)
                logits = jax.lax.dot_general(
                    hw16[sl, :], h16[sl, :], (((1,), (1,)), ((), ())),
                    preferred_element_type=jnp.float32)
                es.append(jnp.exp2(logits))
            ss = []
            for i, b in enumerate(range(g, g + _G)):
                sl = slice(b * _N, (b + 1) * _N)
                pre_aug = jnp.concatenate([pre16[sl, :], ones_blk], axis=1)
                ss.append(_dot(_bf(es[i]), pre_aug))
            for i, b in enumerate(range(g, g + _G)):
                inv = 1.0 / ss[i][:, H:]
                msgs.append(ss[i][:, :H] * inv)
                if t == _ITERS - 1:
                    A_ref[b] = es[i] * jnp.concatenate([inv, inv], axis=1)
        msg = jnp.tanh(jnp.concatenate(msgs, axis=0))
        msg16 = _bf(msg)
        mh16 = jnp.concatenate([msg16, h16], axis=1)
        zr = jnp.tanh(_dot(mh16, Wzr16) + bzr)
        z = 0.5 + 0.5 * zr[:, :H]
        r = 0.5 + 0.5 * zr[:, H:]
        mrh16 = jnp.concatenate([msg16, _bf(r * h)], axis=1)
        htil = jnp.tanh(_dot(mrh16, Wcand16) + bh_ref[t])
        h = h + z * (htil - h)
    pooled = jnp.concatenate(
        [jnp.sum(h[b * _N:(b + 1) * _N, :], axis=0, keepdims=True)
         for b in range(_BB)], axis=0)
    out_ref[...] = jnp.tanh(_dot(_bf(pooled), _bf(Wro_ref[...])) + bro_ref[...])


def kernel(jets, W_emb, b_emb, W_adj, W_msg, b_msg,
           Wz, Uz, bz, Wr, Ur, br, Wh, Uh, bh, W_ro, b_ro):
    B, N, F = jets.shape
    H = _HIDDEN
    x = jets.reshape(B * N, F)

    def rep(ix):
        return pl.BlockSpec(ix, lambda i: (0,) * len(ix))

    out, A = pl.pallas_call(
        _mpnn_kernel,
        grid=(B // _BB,),
        in_specs=[
            pl.BlockSpec((_BB * N, F), lambda i: (i, 0)),
            rep((F + 1, H)),
            rep((1, H)),
            rep((_ITERS, H, H)),
            rep((_ITERS, H, H)),
            rep((_ITERS, 1, H)),
            rep((_ITERS, H, H)), rep((_ITERS, H, H)), rep((_ITERS, 1, H)),
            rep((_ITERS, H, H)), rep((_ITERS, H, H)), rep((_ITERS, 1, H)),
            rep((_ITERS, H, H)), rep((_ITERS, H, H)), rep((_ITERS, 1, H)),
            rep((H, H)),
            rep((1, H)),
        ],
        out_specs=[
            pl.BlockSpec((_BB, H), lambda i: (i, 0)),
            pl.BlockSpec((_BB, N, N), lambda i: (i, 0, 0)),
        ],
        out_shape=[
            jax.ShapeDtypeStruct((B, H), jnp.float32),
            jax.ShapeDtypeStruct((B, N, N), jnp.float32),
        ],
        compiler_params=pltpu.CompilerParams(
            dimension_semantics=("parallel",)),
    )(x, W_emb, b_emb.reshape(1, H),
      W_adj, W_msg, b_msg.reshape(_ITERS, 1, H),
      Wz, Uz, bz.reshape(_ITERS, 1, H),
      Wr, Ur, br.reshape(_ITERS, 1, H),
      Wh, Uh, bh.reshape(_ITERS, 1, H),
      W_ro, b_ro.reshape(1, H))
    return (out, A)
